# baseline (speedup 1.0000x reference)
import functools

import jax
import jax.numpy as jnp
from jax import lax
from jax.experimental import pallas as pl
from jax.experimental.pallas import tpu as pltpu
from jax.experimental.pallas import tpu_sc as plsc

NSEG = 10000
NNZ = 320000
D = 128
HALF = NSEG // 2
CHUNK = 128
NCHUNKS = NNZ // CHUNK
NS = 16
NSLOT = 4
LOOKAHEAD = 3
P1 = 20000
P2 = 1280
TRASH = HALF
ACC_ROWS = HALF + 8
TILE_W = 312
WREM_OFF = NS * TILE_W
WREM = HALF - WREM_OFF
WSIZES = (64, 64, 64, 64, 56)


def _sc_segment_sum(values, ids):
    mesh = plsc.VectorSubcoreMesh(core_axis_name="c", subcore_axis_name="s")

    @functools.partial(
        pl.kernel,
        out_type=jax.ShapeDtypeStruct((NSEG, D), jnp.float32),
        mesh=mesh,
        scratch_types=[
            pltpu.VMEM_SHARED((ACC_ROWS, D), jnp.float32),
            *[pltpu.VMEM((CHUNK, D), jnp.float32) for _ in range(NSLOT)],
            *[pltpu.VMEM((CHUNK,), jnp.int32) for _ in range(NSLOT)],
            pltpu.VMEM((64, D), jnp.float32),
            pltpu.VMEM((16,), jnp.int32),
            pltpu.SMEM((2,), jnp.int32),
            *[pltpu.SemaphoreType.DMA for _ in range(NSLOT)],
            *[pltpu.SemaphoreType.DMA for _ in range(NSLOT)],
        ],
    )
    def k(values_hbm, ids_hbm, out_hbm, acc,
          vb0, vb1, vb2, vb3, ib0, ib1, ib2, ib3, zbuf, cbuf, cnt_smem,
          ls0, ls1, ls2, ls3, ss0, ss1, ss2, ss3):
        c = lax.axis_index("c")
        s = lax.axis_index("s")
        vbufs = (vb0, vb1, vb2, vb3)
        ibufs = (ib0, ib1, ib2, ib3)
        lsems = (ls0, ls1, ls2, ls3)
        ssems = (ss0, ss1, ss2, ss3)

        @pl.loop(0, 64)
        def _zero(i):
            for j in range(D // 16):
                zbuf[i, pl.ds(j * 16, 16)] = jnp.zeros((16,), jnp.float32)

        off = 0
        for sz in WSIZES:
            pltpu.async_copy(
                zbuf.at[pl.ds(0, sz)], acc.at[pl.ds(s * TILE_W + off, sz)],
                ss0)
            off += sz

        @pl.when(s == NS - 1)
        def _zero_rem():
            pltpu.async_copy(
                zbuf.at[pl.ds(0, WREM)], acc.at[pl.ds(WREM_OFF, WREM)], ss0)

        @pl.when(s == 0)
        def _zero_cnt():
            cnt_smem[0] = 0
            cnt_smem[1] = 0

        plsc.subcore_barrier()
        pltpu.sync_copy(ids_hbm.at[pl.ds(s * P1, 16)], cbuf)
        f1 = jnp.where(cbuf[pl.ds(0, 16)][0] < HALF, 1, 0)
        plsc.fetch_and_add(cnt_smem.at[0], f1, subcore_id=0)
        plsc.subcore_barrier()
        r1 = plsc.fetch_and_add(cnt_smem.at[0], 0, subcore_id=0)

        base = jnp.maximum(r1 - 1, 0) * P1
        pltpu.sync_copy(ids_hbm.at[pl.ds(base + s * P2, 16)], cbuf)
        f2 = jnp.where(cbuf[pl.ds(0, 16)][0] < HALF, 1, 0)
        plsc.fetch_and_add(cnt_smem.at[1], f2, subcore_id=0)
        plsc.subcore_barrier()
        r2 = plsc.fetch_and_add(cnt_smem.at[1], 0, subcore_id=0)

        lblo = base + jnp.maximum(r2 - 1, 0) * P2
        lbhi = base + jnp.where(r2 < NS, r2 * P2, P1)

        cs = jnp.where(c == 0, 0, lblo // CHUNK)
        ce = jnp.where(
            c == 0,
            jnp.minimum((lbhi + CHUNK - 1) // CHUNK, NCHUNKS),
            NCHUNKS)
        nk = jnp.maximum(ce - cs - s, 0)
        nk = (nk + NS - 1) // NS
        lo = c * HALF

        def _load(kk, slot):
            @pl.when(kk < nk)
            def _():
                base = (cs + s + NS * kk) * CHUNK
                pltpu.async_copy(
                    values_hbm.at[pl.ds(base, CHUNK)], vbufs[slot],
                    lsems[slot])
                pltpu.async_copy(
                    ids_hbm.at[pl.ds(base, CHUNK)], ibufs[slot], lsems[slot])

        for kk in range(LOOKAHEAD):
            _load(kk, kk)

        off = 0
        for sz in WSIZES:
            pltpu.make_async_copy(
                zbuf.at[pl.ds(0, sz)], acc.at[pl.ds(s * TILE_W + off, sz)],
                ss0).wait()
            off += sz

        @pl.when(s == NS - 1)
        def _zero_rem_wait():
            pltpu.make_async_copy(
                zbuf.at[pl.ds(0, WREM)], acc.at[pl.ds(WREM_OFF, WREM)],
                ss0).wait()

        plsc.subcore_barrier()

        nk4 = ((nk + NSLOT - 1) // NSLOT) * NSLOT

        @pl.loop(0, nk4, step=NSLOT)
        def _body(kk0):
            for b in range(NSLOT):
                vb, ib = vbufs[b], ibufs[b]
                kk = kk0 + b
                nxt = (b + LOOKAHEAD) % NSLOT

                @pl.when(kk < nk)
                def _():
                    pltpu.make_async_copy(
                        values_hbm.at[pl.ds(0, CHUNK)], vb, lsems[b]).wait()
                    pltpu.make_async_copy(
                        ids_hbm.at[pl.ds(0, CHUNK)], ib, lsems[b]).wait()
                    for j in range(CHUNK // 16):
                        iv = ib[pl.ds(j * 16, 16)]
                        loc = iv - lo
                        ok = (loc >= 0) & (loc < HALF)
                        trash = jnp.full((16,), TRASH, jnp.int32)
                        ib[pl.ds(j * 16, 16)] = jnp.where(ok, loc, trash)
                    pltpu.async_copy(vb, acc.at[ib], ssems[b], add=True)

                    @pl.when(kk >= 1)
                    def _():
                        pltpu.make_async_copy(
                            vbufs[nxt], acc.at[ibufs[nxt]], ssems[nxt]).wait()

                    _load(kk + LOOKAHEAD, nxt)

        @pl.when(nk > 0)
        def _drain():
            last = (nk - 1) % NSLOT
            for b in range(NSLOT):
                @pl.when(last == b)
                def _():
                    pltpu.make_async_copy(
                        vbufs[b], acc.at[ibufs[b]], ssems[b]).wait()

        plsc.subcore_barrier()

        pltpu.sync_copy(
            acc.at[pl.ds(s * TILE_W, TILE_W)],
            out_hbm.at[pl.ds(lo + s * TILE_W, TILE_W)])

        @pl.when(s == NS - 1)
        def _wb_rem():
            pltpu.sync_copy(
                acc.at[pl.ds(WREM_OFF, WREM)],
                out_hbm.at[pl.ds(lo + WREM_OFF, WREM)])

    return k(values, ids)


def kernel(values, segment_ids):
    ids = segment_ids.astype(jnp.int32)
    return _sc_segment_sum(values, ids)


# speedup vs baseline: 1.0024x; 1.0024x over previous
import functools

import jax
import jax.numpy as jnp
from jax import lax
from jax.experimental import pallas as pl
from jax.experimental.pallas import tpu as pltpu
from jax.experimental.pallas import tpu_sc as plsc

NSEG = 10000
NNZ = 320000
D = 128
HALF = NSEG // 2
CHUNK = 128
NCHUNKS = NNZ // CHUNK
NS = 16
NSLOT = 4
LOOKAHEAD = 3
FPAD = 2560
FPT = FPAD // NS
TRASH = HALF
ACC_ROWS = HALF + 8
TILE_W = 312
WREM_OFF = NS * TILE_W
WREM = HALF - WREM_OFF
WSIZES = (64, 64, 64, 64, 56)


def _sc_segment_sum(values, ids, firsts):
    mesh = plsc.VectorSubcoreMesh(core_axis_name="c", subcore_axis_name="s")

    @functools.partial(
        pl.kernel,
        out_type=jax.ShapeDtypeStruct((NSEG, D), jnp.float32),
        mesh=mesh,
        scratch_types=[
            pltpu.VMEM_SHARED((ACC_ROWS, D), jnp.float32),
            *[pltpu.VMEM((CHUNK, D), jnp.float32) for _ in range(NSLOT)],
            *[pltpu.VMEM((CHUNK,), jnp.int32) for _ in range(NSLOT)],
            pltpu.VMEM((64, D), jnp.float32),
            pltpu.VMEM((FPT,), jnp.int32),
            pltpu.SMEM((1,), jnp.int32),
            *[pltpu.SemaphoreType.DMA for _ in range(NSLOT)],
            *[pltpu.SemaphoreType.DMA for _ in range(NSLOT)],
            *[pltpu.SemaphoreType.DMA for _ in range(NSLOT)],
        ],
    )
    def k(values_hbm, ids_hbm, firsts_hbm, out_hbm, acc,
          vb0, vb1, vb2, vb3, ib0, ib1, ib2, ib3, zbuf, cbuf, cnt_smem,
          ls0, ls1, ls2, ls3, is0, is1, is2, is3, ss0, ss1, ss2, ss3):
        c = lax.axis_index("c")
        s = lax.axis_index("s")
        vbufs = (vb0, vb1, vb2, vb3)
        ibufs = (ib0, ib1, ib2, ib3)
        lsems = (ls0, ls1, ls2, ls3)
        isems = (is0, is1, is2, is3)
        ssems = (ss0, ss1, ss2, ss3)

        @pl.loop(0, 64)
        def _zero(i):
            for j in range(D // 16):
                zbuf[i, pl.ds(j * 16, 16)] = jnp.zeros((16,), jnp.float32)

        off = 0
        for sz in WSIZES:
            pltpu.async_copy(
                zbuf.at[pl.ds(0, sz)], acc.at[pl.ds(s * TILE_W + off, sz)],
                ss0)
            off += sz

        @pl.when(s == NS - 1)
        def _zero_rem():
            pltpu.async_copy(
                zbuf.at[pl.ds(0, WREM)], acc.at[pl.ds(WREM_OFF, WREM)], ss0)

        @pl.when(s == 0)
        def _zero_cnt():
            cnt_smem[0] = 0

        plsc.subcore_barrier()
        pltpu.sync_copy(firsts_hbm.at[pl.ds(s * FPT, FPT)], cbuf)
        ib0[pl.ds(0, 16)] = jnp.zeros((16,), jnp.int32)

        @pl.loop(0, FPT // 16)
        def _cnt(i):
            iv = cbuf[pl.ds(i * 16, 16)]
            one = jnp.ones((16,), jnp.int32)
            zero = jnp.zeros((16,), jnp.int32)
            ib0[pl.ds(0, 16)] = (
                ib0[pl.ds(0, 16)] + jnp.where(iv < HALF, one, zero))

        cv = ib0[pl.ds(0, 16)]
        my_cnt = cv[0]
        for l in range(1, 16):
            my_cnt = my_cnt + cv[l]
        plsc.fetch_and_add(cnt_smem, my_cnt, subcore_id=0)
        plsc.subcore_barrier()
        c0 = plsc.fetch_and_add(cnt_smem, 0, subcore_id=0)

        cs = jnp.where(c == 0, 0, jnp.maximum(c0 - 1, 0))
        ce = jnp.where(c == 0, c0, NCHUNKS)
        nk = jnp.maximum(ce - cs - s, 0)
        nk = (nk + NS - 1) // NS
        lo = c * HALF

        def _load(kk, slot):
            @pl.when(kk < nk)
            def _():
                base = (cs + s + NS * kk) * CHUNK
                pltpu.async_copy(
                    values_hbm.at[pl.ds(base, CHUNK)], vbufs[slot],
                    lsems[slot])
                pltpu.async_copy(
                    ids_hbm.at[pl.ds(base, CHUNK)], ibufs[slot], isems[slot])

        for kk in range(LOOKAHEAD):
            _load(kk, kk)

        off = 0
        for sz in WSIZES:
            pltpu.make_async_copy(
                zbuf.at[pl.ds(0, sz)], acc.at[pl.ds(s * TILE_W + off, sz)],
                ss0).wait()
            off += sz

        @pl.when(s == NS - 1)
        def _zero_rem_wait():
            pltpu.make_async_copy(
                zbuf.at[pl.ds(0, WREM)], acc.at[pl.ds(WREM_OFF, WREM)],
                ss0).wait()

        plsc.subcore_barrier()

        nk4 = ((nk + NSLOT - 1) // NSLOT) * NSLOT

        @pl.loop(0, nk4, step=NSLOT)
        def _body(kk0):
            for b in range(NSLOT):
                vb, ib = vbufs[b], ibufs[b]
                kk = kk0 + b
                nxt = (b + LOOKAHEAD) % NSLOT

                @pl.when(kk < nk)
                def _():
                    @pl.when(kk >= 1)
                    def _():
                        pltpu.make_async_copy(
                            vbufs[nxt], acc.at[ibufs[nxt]], ssems[nxt]).wait()

                    _load(kk + LOOKAHEAD, nxt)
                    pltpu.make_async_copy(
                        ids_hbm.at[pl.ds(0, CHUNK)], ib, isems[b]).wait()
                    for j in range(CHUNK // 16):
                        iv = ib[pl.ds(j * 16, 16)]
                        loc = iv - lo
                        ok = (loc >= 0) & (loc < HALF)
                        trash = jnp.full((16,), TRASH, jnp.int32)
                        ib[pl.ds(j * 16, 16)] = jnp.where(ok, loc, trash)
                    pltpu.make_async_copy(
                        values_hbm.at[pl.ds(0, CHUNK)], vb, lsems[b]).wait()
                    pltpu.async_copy(vb, acc.at[ib], ssems[b], add=True)

        @pl.when(nk > 0)
        def _drain():
            last = (nk - 1) % NSLOT
            for b in range(NSLOT):
                @pl.when(last == b)
                def _():
                    pltpu.make_async_copy(
                        vbufs[b], acc.at[ibufs[b]], ssems[b]).wait()

        plsc.subcore_barrier()

        pltpu.sync_copy(
            acc.at[pl.ds(s * TILE_W, TILE_W)],
            out_hbm.at[pl.ds(lo + s * TILE_W, TILE_W)])

        @pl.when(s == NS - 1)
        def _wb_rem():
            pltpu.sync_copy(
                acc.at[pl.ds(WREM_OFF, WREM)],
                out_hbm.at[pl.ds(lo + WREM_OFF, WREM)])

    return k(values, ids, firsts)


def kernel(values, segment_ids):
    ids = segment_ids.astype(jnp.int32)
    firsts = jnp.pad(
        ids.reshape(NCHUNKS, CHUNK)[:, 0], (0, FPAD - NCHUNKS),
        constant_values=NSEG)
    return _sc_segment_sum(values, ids, firsts)


# speedup vs baseline: 1.0058x; 1.0034x over previous
import functools

import jax
import jax.numpy as jnp
from jax import lax
from jax.experimental import pallas as pl
from jax.experimental.pallas import tpu as pltpu
from jax.experimental.pallas import tpu_sc as plsc

NSEG = 10000
NNZ = 320000
D = 128
HALF = NSEG // 2
CHUNK = 128
NCHUNKS = NNZ // CHUNK
NS = 16
NSLOT = 4
LOOKAHEAD = 3
FPAD = 2560
FPT = FPAD // NS
TRASH = HALF
ACC_ROWS = HALF + 8
TILE_W = 312
WREM_OFF = NS * TILE_W
WREM = HALF - WREM_OFF
WSIZES = (64, 64, 64, 64, 56)


def _sc_segment_sum(values, ids, firsts):
    mesh = plsc.VectorSubcoreMesh(core_axis_name="c", subcore_axis_name="s")

    @functools.partial(
        pl.kernel,
        out_type=jax.ShapeDtypeStruct((NSEG, D), jnp.float32),
        mesh=mesh,
        scratch_types=[
            pltpu.VMEM_SHARED((ACC_ROWS, D), jnp.float32),
            *[pltpu.VMEM((CHUNK, D), jnp.float32) for _ in range(NSLOT)],
            *[pltpu.VMEM((CHUNK,), jnp.int32) for _ in range(NSLOT)],
            pltpu.VMEM((64, D), jnp.float32),
            pltpu.VMEM((FPT,), jnp.int32),
            pltpu.SMEM((1,), jnp.int32),
            *[pltpu.SemaphoreType.DMA for _ in range(NSLOT)],
            *[pltpu.SemaphoreType.DMA for _ in range(NSLOT)],
        ],
    )
    def k(values_hbm, ids_hbm, firsts_hbm, out_hbm, acc,
          vb0, vb1, vb2, vb3, ib0, ib1, ib2, ib3, zbuf, cbuf, cnt_smem,
          ls0, ls1, ls2, ls3, ss0, ss1, ss2, ss3):
        c = lax.axis_index("c")
        s = lax.axis_index("s")
        vbufs = (vb0, vb1, vb2, vb3)
        ibufs = (ib0, ib1, ib2, ib3)
        lsems = (ls0, ls1, ls2, ls3)
        ssems = (ss0, ss1, ss2, ss3)

        @pl.loop(0, 64)
        def _zero(i):
            for j in range(D // 16):
                zbuf[i, pl.ds(j * 16, 16)] = jnp.zeros((16,), jnp.float32)

        off = 0
        for sz in WSIZES:
            pltpu.async_copy(
                zbuf.at[pl.ds(0, sz)], acc.at[pl.ds(s * TILE_W + off, sz)],
                ss0)
            off += sz

        @pl.when(s == NS - 1)
        def _zero_rem():
            pltpu.async_copy(
                zbuf.at[pl.ds(0, WREM)], acc.at[pl.ds(WREM_OFF, WREM)], ss0)

        @pl.when(s == 0)
        def _zero_cnt():
            cnt_smem[0] = 0

        plsc.subcore_barrier()
        pltpu.sync_copy(firsts_hbm.at[pl.ds(s * FPT, FPT)], cbuf)
        ib0[pl.ds(0, 16)] = jnp.zeros((16,), jnp.int32)

        @pl.loop(0, FPT // 16)
        def _cnt(i):
            iv = cbuf[pl.ds(i * 16, 16)]
            one = jnp.ones((16,), jnp.int32)
            zero = jnp.zeros((16,), jnp.int32)
            ib0[pl.ds(0, 16)] = (
                ib0[pl.ds(0, 16)] + jnp.where(iv < HALF, one, zero))

        cv = ib0[pl.ds(0, 16)]
        my_cnt = cv[0]
        for l in range(1, 16):
            my_cnt = my_cnt + cv[l]
        plsc.fetch_and_add(cnt_smem, my_cnt, subcore_id=0)
        plsc.subcore_barrier()
        c0 = plsc.fetch_and_add(cnt_smem, 0, subcore_id=0)

        cs = jnp.where(c == 0, 0, jnp.maximum(c0 - 1, 0))
        ce = jnp.where(c == 0, c0, NCHUNKS)
        nk = jnp.maximum(ce - cs - s, 0)
        nk = (nk + NS - 1) // NS
        lo = c * HALF

        def _load(kk, slot):
            @pl.when(kk < nk)
            def _():
                base = (cs + s + NS * kk) * CHUNK
                pltpu.async_copy(
                    values_hbm.at[pl.ds(base, CHUNK)], vbufs[slot],
                    lsems[slot])
                pltpu.async_copy(
                    ids_hbm.at[pl.ds(base, CHUNK)], ibufs[slot], lsems[slot])

        for kk in range(LOOKAHEAD):
            _load(kk, kk)

        off = 0
        for sz in WSIZES:
            pltpu.make_async_copy(
                zbuf.at[pl.ds(0, sz)], acc.at[pl.ds(s * TILE_W + off, sz)],
                ss0).wait()
            off += sz

        @pl.when(s == NS - 1)
        def _zero_rem_wait():
            pltpu.make_async_copy(
                zbuf.at[pl.ds(0, WREM)], acc.at[pl.ds(WREM_OFF, WREM)],
                ss0).wait()

        plsc.subcore_barrier()

        nk4 = ((nk + NSLOT - 1) // NSLOT) * NSLOT

        @pl.loop(0, nk4, step=NSLOT)
        def _body(kk0):
            for b in range(NSLOT):
                vb, ib = vbufs[b], ibufs[b]
                kk = kk0 + b
                nxt = (b + LOOKAHEAD) % NSLOT

                @pl.when(kk < nk)
                def _():
                    pltpu.make_async_copy(
                        values_hbm.at[pl.ds(0, CHUNK)], vb, lsems[b]).wait()
                    pltpu.make_async_copy(
                        ids_hbm.at[pl.ds(0, CHUNK)], ib, lsems[b]).wait()
                    for j in range(CHUNK // 16):
                        iv = ib[pl.ds(j * 16, 16)]
                        loc = iv - lo
                        ok = (loc >= 0) & (loc < HALF)
                        trash = jnp.full((16,), TRASH, jnp.int32)
                        ib[pl.ds(j * 16, 16)] = jnp.where(ok, loc, trash)
                    pltpu.async_copy(vb, acc.at[ib], ssems[b], add=True)

                    @pl.when(kk >= 1)
                    def _():
                        pltpu.make_async_copy(
                            vbufs[nxt], acc.at[ibufs[nxt]], ssems[nxt]).wait()

                    _load(kk + LOOKAHEAD, nxt)

        @pl.when(nk > 0)
        def _drain():
            last = (nk - 1) % NSLOT
            for b in range(NSLOT):
                @pl.when(last == b)
                def _():
                    pltpu.make_async_copy(
                        vbufs[b], acc.at[ibufs[b]], ssems[b]).wait()

        plsc.subcore_barrier()

        pltpu.sync_copy(
            acc.at[pl.ds(s * TILE_W, TILE_W)],
            out_hbm.at[pl.ds(lo + s * TILE_W, TILE_W)])

        @pl.when(s == NS - 1)
        def _wb_rem():
            pltpu.sync_copy(
                acc.at[pl.ds(WREM_OFF, WREM)],
                out_hbm.at[pl.ds(lo + WREM_OFF, WREM)])

    return k(values, ids, firsts)


def kernel(values, segment_ids):
    ids = segment_ids.astype(jnp.int32)
    firsts = jnp.pad(
        ids.reshape(NCHUNKS, CHUNK)[:, 0], (0, FPAD - NCHUNKS),
        constant_values=NSEG)
    return _sc_segment_sum(values, ids, firsts)
